# baseline (speedup 1.0000x reference)
import jax
import jax.numpy as jnp
from jax.experimental import pallas as pl

_ROW_BLOCK = 2000


def _linear_kernel(f_ref, w_ref, b_ref, o_ref):
    o_ref[...] = jax.lax.dot_general(
        f_ref[...], w_ref[...],
        dimension_numbers=(((1,), (1,)), ((), ())),
        preferred_element_type=jnp.float32,
    ) + b_ref[...]


def kernel(feats, W, b):
    M, K = feats.shape
    N = W.shape[0]
    b2 = b.reshape(1, N)
    return pl.pallas_call(
        _linear_kernel,
        grid=(M // _ROW_BLOCK,),
        in_specs=[
            pl.BlockSpec((_ROW_BLOCK, K), lambda i: (i, 0)),
            pl.BlockSpec((N, K), lambda i: (0, 0)),
            pl.BlockSpec((1, N), lambda i: (0, 0)),
        ],
        out_specs=pl.BlockSpec((_ROW_BLOCK, N), lambda i: (i, 0)),
        out_shape=jax.ShapeDtypeStruct((M, N), jnp.float32),
    )(feats, W, b2)
